# baseline (speedup 1.0000x reference)
import functools

import jax
import jax.numpy as jnp
from jax import lax
from jax.experimental import pallas as pl
from jax.experimental.pallas import tpu as pltpu
from jax.experimental.pallas import tpu_sc as plsc

_B = 16384
_D = 64
_D3 = 192
_MARGIN = 1.0
_NC, _NS, _L = 2, 16, 16
_NW = _NC * _NS
_PER_W = _B // _NW
_CHUNK = 64
_NCHUNK = _PER_W // _CHUNK
_NGRP = _CHUNK // _L


def _body(pos_hbm, dn_hbm, idx_hbm, w_hbm, wt_hbm, out_hbm,
          idx_v, rows_v, tail_v, pos_v, dn_v,
          out_v, tp_v, sem, sem2):
    wid = lax.axis_index("s") * _NC + lax.axis_index("c")
    base = wid * _PER_W
    pltpu.sync_copy(idx_hbm.at[pl.ds(base, _PER_W)], idx_v)
    pltpu.sync_copy(dn_hbm.at[pl.ds(base, _PER_W)], dn_v)

    def chunk_body(ci, carry):
        cbase = ci * _CHUNK
        rows = idx_v.at[pl.ds(cbase, _CHUNK)]
        cp0 = pltpu.async_copy(w_hbm.at[rows, pl.ds(0, 128)], rows_v, sem)
        cp1 = pltpu.async_copy(wt_hbm.at[rows], tail_v, sem2)
        pltpu.sync_copy(pos_hbm.at[pl.ds(base + cbase, _CHUNK)], pos_v)
        cp0.wait()
        cp1.wait()

        def grp_body(g, gcarry):
            e0 = g * _L
            for e in range(_L):
                row = e0 + e
                sp = [None] * 4
                for j in range(4):
                    w0 = rows_v[row, pl.ds(j * _L, _L)]
                    w1 = rows_v[row, pl.ds(_D + j * _L, _L)]
                    w2 = tail_v[row, pl.ds(j * _L, _L)]
                    p0 = pos_v[row, pl.ds(j * _L, _L)]
                    p1 = pos_v[row, pl.ds(_D + j * _L, _L)]
                    p2 = pos_v[row, pl.ds(2 * _D + j * _L, _L)]
                    sp[j] = jnp.abs(w0 * p0 + w1 * p1 + w2 * p2)
                tp_v[pl.ds(e * _L, _L)] = (sp[0] + sp[1]) + (sp[2] + sp[3])
            dp = jnp.zeros((_L,), jnp.float32)
            lane = lax.iota(jnp.int32, _L) * _L
            for c in range(_L):
                col = lane + c
                dp = dp + plsc.load_gather(tp_v, [col])
            dn = dn_v[pl.ds(cbase + e0, _L)]
            loss = jnp.maximum(dp - dn + _MARGIN, 0.0)
            out_v[pl.ds(cbase + e0, _L)] = loss
            return gcarry

        lax.fori_loop(0, _NGRP, grp_body, 0)
        return carry

    lax.fori_loop(0, _NCHUNK, chunk_body, 0)
    pltpu.sync_copy(out_v, out_hbm.at[pl.ds(base, _PER_W)])


_sc_call = functools.partial(
    pl.kernel,
    mesh=plsc.VectorSubcoreMesh(core_axis_name="c", subcore_axis_name="s"),
    out_type=jax.ShapeDtypeStruct((_B,), jnp.float32),
    compiler_params=pltpu.CompilerParams(
        use_tc_tiling_on_sc=True, needs_layout_passes=False),
    scratch_types=[
        pltpu.VMEM((_PER_W,), jnp.int32),
        pltpu.VMEM((_CHUNK, 128), jnp.float32),
        pltpu.VMEM((_CHUNK, 128), jnp.float32),
        pltpu.VMEM((_CHUNK, _D3), jnp.float32),
        pltpu.VMEM((_PER_W,), jnp.float32),
        pltpu.VMEM((_PER_W,), jnp.float32),
        pltpu.VMEM((_L * _L,), jnp.float32),
        pltpu.SemaphoreType.DMA,
        pltpu.SemaphoreType.DMA,
    ],
)(_body)


_NROWS = 100000
_TAIL_BLK = 1000


def _tail_body(w_ref, o_ref):
    o_ref[:, :_D] = w_ref[:, :_D]


_w_tail = pl.pallas_call(
    _tail_body,
    grid=(_NROWS // _TAIL_BLK,),
    in_specs=[pl.BlockSpec((_TAIL_BLK, 128), lambda i: (i, 1))],
    out_specs=pl.BlockSpec((_TAIL_BLK, 128), lambda i: (i, 0)),
    out_shape=jax.ShapeDtypeStruct((_NROWS, 128), jnp.float32),
)


_PN_BLK = 256


def _posneg_body(p_ref, n_ref, p2_ref, dn_ref):
    p2_ref[:, pl.ds(0, _D)] = p_ref[:, 0, :]
    p2_ref[:, pl.ds(_D, _D)] = p_ref[:, 1, :]
    p2_ref[:, pl.ds(2 * _D, _D)] = p_ref[:, 2, :]
    s = n_ref[:, 0, :] + n_ref[:, 1, :] + n_ref[:, 2, :]
    dn_ref[...] = jnp.sum(jnp.abs(s), axis=1)


_posneg = pl.pallas_call(
    _posneg_body,
    grid=(_B // _PN_BLK,),
    in_specs=[pl.BlockSpec((_PN_BLK, 3, _D), lambda i: (i, 0, 0)),
              pl.BlockSpec((_PN_BLK, 3, _D), lambda i: (i, 0, 0))],
    out_specs=[pl.BlockSpec((_PN_BLK, _D3), lambda i: (i, 0)),
               pl.BlockSpec((_PN_BLK,), lambda i: (i,))],
    out_shape=[jax.ShapeDtypeStruct((_B, _D3), jnp.float32),
               jax.ShapeDtypeStruct((_B,), jnp.float32)],
)


def kernel(pos_triples, neg_triples, order_hrt, W):
    pos2, dn = _posneg(pos_triples, neg_triples)
    return _sc_call(pos2, dn, order_hrt, W, _w_tail(W))


# speedup vs baseline: 1.4571x; 1.4571x over previous
import functools

import jax
import jax.numpy as jnp
from jax import lax
from jax.experimental import pallas as pl
from jax.experimental.pallas import tpu as pltpu
from jax.experimental.pallas import tpu_sc as plsc

_B = 16384
_D = 64
_D3 = 192
_MARGIN = 1.0
_NC, _NS, _L = 2, 16, 16
_NW = _NC * _NS
_PER_W = _B // _NW
_CHUNK = 64
_HALF = 50000
_NCHUNK = _PER_W // _CHUNK
_NGRP = _CHUNK // _L


def _body(pos_hbm, neg_hbm, idx_hbm, w_hbm, wt_hbm, out_hbm,
          idx_v, idxb_v, off_v, rows_v, tail_v, pos_v, neg_v,
          out_v, tp_v, tn_v, sem, sem2):
    wid = lax.axis_index("s") * _NC + lax.axis_index("c")
    base = wid * _PER_W
    pltpu.sync_copy(idx_hbm.at[pl.ds(base, _PER_W)], idx_v)

    def pre_body(t, c):
        tb = pl.ds(t * _L, _L)
        v = idx_v[tb]
        m = v >= _HALF
        idxb_v[tb] = jnp.where(m, v - _HALF, v)
        off_v[tb] = jnp.where(m, _D, 0)
        return c

    lax.fori_loop(0, _PER_W // _L, pre_body, 0)

    def chunk_body(ci, carry):
        cbase = ci * _CHUNK
        rows = idx_v.at[pl.ds(cbase, _CHUNK)]
        brow = pl.ds(base + cbase, _CHUNK)
        cp0 = pltpu.async_copy(w_hbm.at[rows, pl.ds(0, 128)], rows_v, sem)
        cp1 = pltpu.async_copy(wt_hbm.at[idxb_v.at[pl.ds(cbase, _CHUNK)]],
                               tail_v, sem2)
        pltpu.sync_copy(pos_hbm.at[brow], pos_v)
        pltpu.sync_copy(neg_hbm.at[brow], neg_v)
        cp0.wait()
        cp1.wait()

        def grp_body(g, gcarry):
            e0 = g * _L
            zvec = jnp.zeros((_L,), jnp.int32)
            lidx = lax.iota(jnp.int32, _L)
            for e in range(_L):
                row = e0 + e
                offv = plsc.load_gather(off_v, [zvec + (cbase + row)])
                rowv = zvec + row
                sp = [None] * 4
                sn = [None] * 4
                for j in range(4):
                    w0 = rows_v[row, pl.ds(j * _L, _L)]
                    w1 = rows_v[row, pl.ds(_D + j * _L, _L)]
                    w2 = plsc.load_gather(tail_v, [rowv, lidx + j * _L + offv])
                    p0 = pos_v[row, pl.ds(j * _L, _L)]
                    p1 = pos_v[row, pl.ds(_D + j * _L, _L)]
                    p2 = pos_v[row, pl.ds(2 * _D + j * _L, _L)]
                    n0 = neg_v[row, pl.ds(j * _L, _L)]
                    n1 = neg_v[row, pl.ds(_D + j * _L, _L)]
                    n2 = neg_v[row, pl.ds(2 * _D + j * _L, _L)]
                    sp[j] = jnp.abs(w0 * p0 + w1 * p1 + w2 * p2)
                    sn[j] = jnp.abs(n0 + n1 + n2)
                tp_v[pl.ds(e * _L, _L)] = (sp[0] + sp[1]) + (sp[2] + sp[3])
                tn_v[pl.ds(e * _L, _L)] = (sn[0] + sn[1]) + (sn[2] + sn[3])
            zero = jnp.zeros((_L,), jnp.float32)
            dp = zero
            dn = zero
            lane = lax.iota(jnp.int32, _L) * _L
            for c in range(_L):
                col = lane + c
                dp = dp + plsc.load_gather(tp_v, [col])
                dn = dn + plsc.load_gather(tn_v, [col])
            loss = jnp.maximum(dp - dn + _MARGIN, 0.0)
            out_v[pl.ds(cbase + e0, _L)] = loss
            return gcarry

        lax.fori_loop(0, _NGRP, grp_body, 0)
        return carry

    lax.fori_loop(0, _NCHUNK, chunk_body, 0)
    pltpu.sync_copy(out_v, out_hbm.at[pl.ds(base, _PER_W)])


_sc_call = functools.partial(
    pl.kernel,
    mesh=plsc.VectorSubcoreMesh(core_axis_name="c", subcore_axis_name="s"),
    out_type=jax.ShapeDtypeStruct((_B,), jnp.float32),
    compiler_params=pltpu.CompilerParams(
        use_tc_tiling_on_sc=True, needs_layout_passes=False),
    scratch_types=[
        pltpu.VMEM((_PER_W,), jnp.int32),
        pltpu.VMEM((_PER_W,), jnp.int32),
        pltpu.VMEM((_PER_W,), jnp.int32),
        pltpu.VMEM((_CHUNK, 128), jnp.float32),
        pltpu.VMEM((_CHUNK, 128), jnp.float32),
        pltpu.VMEM((_CHUNK, _D3), jnp.float32),
        pltpu.VMEM((_CHUNK, _D3), jnp.float32),
        pltpu.VMEM((_PER_W,), jnp.float32),
        pltpu.VMEM((_L * _L,), jnp.float32),
        pltpu.VMEM((_L * _L,), jnp.float32),
        pltpu.SemaphoreType.DMA,
        pltpu.SemaphoreType.DMA,
    ],
)(_body)


_NROWS = 100000
_TAIL_BLK = 1000
_NBLK_HALF = _HALF // _TAIL_BLK


def _tail_body(a_ref, b_ref, o_ref):
    o_ref[:, :_D] = a_ref[:, :_D]
    o_ref[:, _D:] = b_ref[:, :_D]


_w_tail = pl.pallas_call(
    _tail_body,
    grid=(_NBLK_HALF,),
    in_specs=[pl.BlockSpec((_TAIL_BLK, 128), lambda i: (i, 1)),
              pl.BlockSpec((_TAIL_BLK, 128), lambda i: (i + _NBLK_HALF, 1))],
    out_specs=pl.BlockSpec((_TAIL_BLK, 128), lambda i: (i, 0)),
    out_shape=jax.ShapeDtypeStruct((_HALF, 128), jnp.float32),
)


def kernel(pos_triples, neg_triples, order_hrt, W):
    pos2 = pos_triples.reshape(_B, _D3)
    neg2 = neg_triples.reshape(_B, _D3)
    return _sc_call(pos2, neg2, order_hrt, W, _w_tail(W, W))


# speedup vs baseline: 1.4879x; 1.0211x over previous
import functools

import jax
import jax.numpy as jnp
from jax import lax
from jax.experimental import pallas as pl
from jax.experimental.pallas import tpu as pltpu
from jax.experimental.pallas import tpu_sc as plsc

_B = 16384
_D = 64
_D3 = 192
_MARGIN = 1.0
_NC, _NS, _L = 2, 16, 16
_NW = _NC * _NS
_PER_W = _B // _NW
_CHUNK = 128
_HALF = 50000
_NCHUNK = _PER_W // _CHUNK
_NGRP = _CHUNK // _L


def _body(pos_hbm, neg_hbm, idx_hbm, w_hbm, wt_hbm, out_hbm,
          idx_v, idxb_v, off_v, rows_v, tail_v, pos_v, neg_v,
          out_v, tp_v, tn_v, sem, sem2):
    wid = lax.axis_index("s") * _NC + lax.axis_index("c")
    base = wid * _PER_W
    pltpu.sync_copy(idx_hbm.at[pl.ds(base, _PER_W)], idx_v)

    def pre_body(t, c):
        tb = pl.ds(t * _L, _L)
        v = idx_v[tb]
        m = v >= _HALF
        idxb_v[tb] = jnp.where(m, v - _HALF, v)
        off_v[tb] = jnp.where(m, _D, 0)
        return c

    lax.fori_loop(0, _PER_W // _L, pre_body, 0)

    def chunk_body(ci, carry):
        cbase = ci * _CHUNK
        rows = idx_v.at[pl.ds(cbase, _CHUNK)]
        brow = pl.ds(base + cbase, _CHUNK)
        cp0 = pltpu.async_copy(w_hbm.at[rows, pl.ds(0, 128)], rows_v, sem)
        cp1 = pltpu.async_copy(wt_hbm.at[idxb_v.at[pl.ds(cbase, _CHUNK)]],
                               tail_v, sem2)
        pltpu.sync_copy(pos_hbm.at[brow], pos_v)
        pltpu.sync_copy(neg_hbm.at[brow], neg_v)
        cp0.wait()
        cp1.wait()

        def grp_body(g, gcarry):
            e0 = g * _L
            zvec = jnp.zeros((_L,), jnp.int32)
            lidx = lax.iota(jnp.int32, _L)
            for e in range(_L):
                row = e0 + e
                offv = plsc.load_gather(off_v, [zvec + (cbase + row)])
                rowv = zvec + row
                sp = [None] * 4
                sn = [None] * 4
                for j in range(4):
                    w0 = rows_v[row, pl.ds(j * _L, _L)]
                    w1 = rows_v[row, pl.ds(_D + j * _L, _L)]
                    w2 = plsc.load_gather(tail_v, [rowv, lidx + j * _L + offv])
                    p0 = pos_v[row, pl.ds(j * _L, _L)]
                    p1 = pos_v[row, pl.ds(_D + j * _L, _L)]
                    p2 = pos_v[row, pl.ds(2 * _D + j * _L, _L)]
                    n0 = neg_v[row, pl.ds(j * _L, _L)]
                    n1 = neg_v[row, pl.ds(_D + j * _L, _L)]
                    n2 = neg_v[row, pl.ds(2 * _D + j * _L, _L)]
                    sp[j] = jnp.abs(w0 * p0 + w1 * p1 + w2 * p2)
                    sn[j] = jnp.abs(n0 + n1 + n2)
                tp_v[pl.ds(e * _L, _L)] = (sp[0] + sp[1]) + (sp[2] + sp[3])
                tn_v[pl.ds(e * _L, _L)] = (sn[0] + sn[1]) + (sn[2] + sn[3])
            zero = jnp.zeros((_L,), jnp.float32)
            dp = zero
            dn = zero
            lane = lax.iota(jnp.int32, _L) * _L
            for c in range(_L):
                col = lane + c
                dp = dp + plsc.load_gather(tp_v, [col])
                dn = dn + plsc.load_gather(tn_v, [col])
            loss = jnp.maximum(dp - dn + _MARGIN, 0.0)
            out_v[pl.ds(cbase + e0, _L)] = loss
            return gcarry

        lax.fori_loop(0, _NGRP, grp_body, 0)
        return carry

    lax.fori_loop(0, _NCHUNK, chunk_body, 0)
    pltpu.sync_copy(out_v, out_hbm.at[pl.ds(base, _PER_W)])


_sc_call = functools.partial(
    pl.kernel,
    mesh=plsc.VectorSubcoreMesh(core_axis_name="c", subcore_axis_name="s"),
    out_type=jax.ShapeDtypeStruct((_B,), jnp.float32),
    compiler_params=pltpu.CompilerParams(
        use_tc_tiling_on_sc=True, needs_layout_passes=False),
    scratch_types=[
        pltpu.VMEM((_PER_W,), jnp.int32),
        pltpu.VMEM((_PER_W,), jnp.int32),
        pltpu.VMEM((_PER_W,), jnp.int32),
        pltpu.VMEM((_CHUNK, 128), jnp.float32),
        pltpu.VMEM((_CHUNK, 128), jnp.float32),
        pltpu.VMEM((_CHUNK, _D3), jnp.float32),
        pltpu.VMEM((_CHUNK, _D3), jnp.float32),
        pltpu.VMEM((_PER_W,), jnp.float32),
        pltpu.VMEM((_L * _L,), jnp.float32),
        pltpu.VMEM((_L * _L,), jnp.float32),
        pltpu.SemaphoreType.DMA,
        pltpu.SemaphoreType.DMA,
    ],
)(_body)


_NROWS = 100000
_TAIL_BLK = 1000
_NBLK_HALF = _HALF // _TAIL_BLK


def _tail_body(a_ref, b_ref, o_ref):
    o_ref[:, :_D] = a_ref[:, :_D]
    o_ref[:, _D:] = b_ref[:, :_D]


_w_tail = pl.pallas_call(
    _tail_body,
    grid=(_NBLK_HALF,),
    in_specs=[pl.BlockSpec((_TAIL_BLK, 128), lambda i: (i, 1)),
              pl.BlockSpec((_TAIL_BLK, 128), lambda i: (i + _NBLK_HALF, 1))],
    out_specs=pl.BlockSpec((_TAIL_BLK, 128), lambda i: (i, 0)),
    out_shape=jax.ShapeDtypeStruct((_HALF, 128), jnp.float32),
)


def kernel(pos_triples, neg_triples, order_hrt, W):
    pos2 = pos_triples.reshape(_B, _D3)
    neg2 = neg_triples.reshape(_B, _D3)
    return _sc_call(pos2, neg2, order_hrt, W, _w_tail(W, W))
